# Initial kernel scaffold; baseline (speedup 1.0000x reference)
import jax
import jax.numpy as jnp
from jax.experimental import pallas as pl


def kernel(x, edge_index, W0, b0, W1, b1, W2, b2, W3, b3, Wo, bo):
    raise NotImplementedError("write your pallas kernel here")


# speedup vs baseline: 3.2497x; 3.2497x over previous
import functools

import jax
import jax.numpy as jnp
from jax import lax
from jax.experimental import pallas as pl
from jax.experimental.pallas import tpu as pltpu
from jax.experimental.pallas import tpu_sc as plsc

N = 10000
E = 320000
D = 128
EPS = 1e-07

NP = 10240
EP = 327680
NW = 32
CHUNKS = 80
CW = 128
RPT = 640
RB = 1024


def _sc_segsum(r, src, dst):
    mesh = plsc.VectorSubcoreMesh(core_axis_name="c", subcore_axis_name="s")

    @functools.partial(
        pl.kernel,
        out_type=jax.ShapeDtypeStruct((2, NP, D), jnp.float32),
        mesh=mesh,
        scratch_types=[
            pltpu.VMEM((CHUNKS, CW), jnp.int32),
            pltpu.VMEM((CHUNKS, CW), jnp.int32),
            pltpu.VMEM((CW, D), jnp.float32),
            pltpu.VMEM_SHARED((NP, D), jnp.float32),
        ],
    )
    def k(r_hbm, src_hbm, dst_hbm, out_hbm, src_v, dst_v, rows_v, aggr_sh):
        c = lax.axis_index("c")
        s = lax.axis_index("s")
        w = c * 16 + s

        zero16 = jnp.zeros((16,), jnp.float32)

        @pl.loop(0, CW)
        def _(i):
            @pl.loop(0, D, step=16)
            def _(j):
                rows_v[i, pl.ds(j, 16)] = zero16

        @pl.loop(0, RPT, step=CW)
        def _(k0):
            pltpu.sync_copy(rows_v, aggr_sh.at[pl.ds(s * RPT + k0, CW)])

        pltpu.sync_copy(src_hbm.at[w], src_v)
        pltpu.sync_copy(dst_hbm.at[w], dst_v)

        plsc.subcore_barrier()

        @pl.loop(0, CHUNKS)
        def _(g):
            pltpu.sync_copy(r_hbm.at[src_v.at[g]], rows_v)
            pltpu.sync_copy(rows_v, aggr_sh.at[dst_v.at[g]], add=True)

        plsc.subcore_barrier()

        pltpu.sync_copy(aggr_sh.at[pl.ds(s * RPT, RPT)],
                        out_hbm.at[c, pl.ds(s * RPT, RPT)])

    return k(r, src, dst)


def _relu_eps_body(x_ref, r_ref):
    r_ref[...] = jnp.maximum(x_ref[...], 0.0) + EPS


def _relu_eps(xp):
    return pl.pallas_call(
        _relu_eps_body,
        grid=(NP // RB,),
        in_specs=[pl.BlockSpec((RB, D), lambda i: (i, 0))],
        out_specs=pl.BlockSpec((RB, D), lambda i: (i, 0)),
        out_shape=jax.ShapeDtypeStruct((NP, D), jnp.float32),
    )(xp)


def _update_body(agg_ref, h_ref, w_ref, b_ref, hn_ref, rn_ref):
    t = agg_ref[0] + agg_ref[1] + h_ref[...]
    hn = jnp.dot(t, w_ref[...], preferred_element_type=jnp.float32) + b_ref[...]
    hn_ref[...] = hn
    rn_ref[...] = jnp.maximum(hn, 0.0) + EPS


def _update(agg, h, W, b):
    return pl.pallas_call(
        _update_body,
        grid=(NP // RB,),
        in_specs=[
            pl.BlockSpec((2, RB, D), lambda i: (0, i, 0)),
            pl.BlockSpec((RB, D), lambda i: (i, 0)),
            pl.BlockSpec((D, D), lambda i: (0, 0)),
            pl.BlockSpec((1, D), lambda i: (0, 0)),
        ],
        out_specs=[
            pl.BlockSpec((RB, D), lambda i: (i, 0)),
            pl.BlockSpec((RB, D), lambda i: (i, 0)),
        ],
        out_shape=[
            jax.ShapeDtypeStruct((NP, D), jnp.float32),
            jax.ShapeDtypeStruct((NP, D), jnp.float32),
        ],
    )(agg, h, W, b)


def _final_body(agg_ref, h_ref, w_ref, b_ref, wo_ref, bo_ref, o_ref):
    t = agg_ref[0] + agg_ref[1] + h_ref[...]
    hn = jnp.dot(t, w_ref[...], preferred_element_type=jnp.float32) + b_ref[...]
    o_ref[...] = jnp.dot(hn, wo_ref[...],
                         preferred_element_type=jnp.float32) + bo_ref[...]


def _final(agg, h, W, b, Wo, bo):
    return pl.pallas_call(
        _final_body,
        grid=(NP // RB,),
        in_specs=[
            pl.BlockSpec((2, RB, D), lambda i: (0, i, 0)),
            pl.BlockSpec((RB, D), lambda i: (i, 0)),
            pl.BlockSpec((D, D), lambda i: (0, 0)),
            pl.BlockSpec((1, D), lambda i: (0, 0)),
            pl.BlockSpec((D, D), lambda i: (0, 0)),
            pl.BlockSpec((1, D), lambda i: (0, 0)),
        ],
        out_specs=pl.BlockSpec((RB, D), lambda i: (i, 0)),
        out_shape=jax.ShapeDtypeStruct((NP, D), jnp.float32),
    )(agg, h, W, b, Wo, bo)


def kernel(x, edge_index, W0, b0, W1, b1, W2, b2, W3, b3, Wo, bo):
    ei = edge_index.astype(jnp.int32)
    src = jnp.concatenate([ei[0], jnp.zeros((EP - E,), jnp.int32)])
    dst = jnp.concatenate([ei[1], jnp.full((EP - E,), N, jnp.int32)])
    src = src.reshape(NW, CHUNKS, CW)
    dst = dst.reshape(NW, CHUNKS, CW)

    xp = jnp.pad(x, ((0, NP - N), (0, 0)))

    h = xp
    r = _relu_eps(xp)
    for W, b in ((W0, b0), (W1, b1), (W2, b2)):
        agg = _sc_segsum(r, src, dst)
        h, r = _update(agg, h, W, b.reshape(1, D))
    agg = _sc_segsum(r, src, dst)
    out = _final(agg, h, W3, b3.reshape(1, D), Wo, bo.reshape(1, D))
    return out[:N]
